# baseline (speedup 1.0000x reference)
import functools

import jax
import jax.numpy as jnp
from jax import lax
from jax.experimental import pallas as pl
from jax.experimental.pallas import tpu as pltpu
from jax.experimental.pallas import tpu_sc as plsc

VOCAB = 100000
DIM = 512
B = 16384

NC = 2
NS = 16
NW = NC * NS
BPW = B // NW
CHUNK = 64
NCHUNK = BPW // CHUNK
GROUPS = CHUNK // 16
UNROLL = 8


def _rsqrt_nr(x):
    i = plsc.bitcast(x, jnp.int32)
    i = jnp.int32(0x5F3759DF) - lax.shift_right_arithmetic(i, 1)
    y = plsc.bitcast(i, jnp.float32)
    half_x = x * jnp.float32(0.5)
    for _ in range(3):
        y = y * (jnp.float32(1.5) - half_x * y * y)
    return y


@functools.partial(
    pl.kernel,
    out_type=jax.ShapeDtypeStruct((B, DIM), jnp.float32),
    mesh=plsc.VectorSubcoreMesh(core_axis_name="c", subcore_axis_name="s"),
    compiler_params=pltpu.CompilerParams(
        use_tc_tiling_on_sc=False, needs_layout_passes=False
    ),
    scratch_types=[
        pltpu.VMEM((BPW,), jnp.int32),
        pltpu.VMEM((CHUNK, DIM), jnp.float32),
        pltpu.SemaphoreType.DMA,
    ],
)
def _encode(table_hbm, pids_hbm, out_hbm, idx_v, rows_v, sem):
    cid = lax.axis_index("c")
    sid = lax.axis_index("s")
    wid = sid * NC + cid
    base = wid * BPW
    pltpu.sync_copy(pids_hbm.at[pl.ds(base, BPW)], idx_v)
    for ch in range(NCHUNK):
        pltpu.async_copy(
            table_hbm.at[idx_v.at[pl.ds(ch * CHUNK, CHUNK)]], rows_v, sem
        ).wait()
        for g in range(GROUPS):
            rows16 = lax.iota(jnp.int32, 16) + jnp.int32(g * 16)

            def ssq_step(j, acc, rows16=rows16):
                for k in range(UNROLL):
                    col = jnp.full((16,), j * UNROLL + k, jnp.int32)
                    v = plsc.load_gather(rows_v, [rows16, col])
                    acc = acc + v * v
                return acc

            ssq = lax.fori_loop(
                0, DIM // UNROLL, ssq_step, jnp.zeros((16,), jnp.float32)
            )
            inv = _rsqrt_nr(ssq)

            def scale_step(j, carry, rows16=rows16, inv=inv):
                for k in range(UNROLL):
                    col = jnp.full((16,), j * UNROLL + k, jnp.int32)
                    v = plsc.load_gather(rows_v, [rows16, col])
                    plsc.store_scatter(rows_v, [rows16, col], v * inv)
                return carry

            lax.fori_loop(0, DIM // UNROLL, scale_step, jnp.int32(0))
        pltpu.sync_copy(rows_v, out_hbm.at[pl.ds(base + ch * CHUNK, CHUNK)])


def kernel(classifier_weights, pids):
    return _encode(classifier_weights, pids.astype(jnp.int32))
